# Initial kernel scaffold; baseline (speedup 1.0000x reference)
import jax
import jax.numpy as jnp
from jax.experimental import pallas as pl


def kernel(node_features, edge_index, graph_index, params):
    raise NotImplementedError("write your pallas kernel here")


# speedup vs baseline: 5.3602x; 5.3602x over previous
import functools

import jax
import jax.numpy as jnp
from jax import lax
from jax.experimental import pallas as pl
from jax.experimental.pallas import tpu as pltpu
from jax.experimental.pallas import tpu_sc as plsc

N = 10000
E = 160000
DIM = 256
HALF = 128
G = 64

NTILE = 16
EPT = E // NTILE
K = 200
CH = EPT // K
IDXG = 10
RPT = 624
WB = 104
NWB = RPT // WB
TAIL = N - NTILE * RPT

NB = 10
BR = N // NB


def _sc_agg_body(h0_hbm, h1_hbm, src_hbm, dst_hbm, out0_hbm, out1_hbm,
                 sidx, didx, rows, wb, acc, sem):
    c = lax.axis_index("c")
    s = lax.axis_index("s")
    base = s * RPT

    def slab_copy(load, store):
        for kk in range(NWB):
            r0 = base + kk * WB
            load(pl.ds(r0, WB), None)
            store(pl.ds(r0, WB), None)

        @pl.when(s == NTILE - 1)
        def _():
            load(pl.ds(NTILE * RPT, TAIL), pl.ds(0, TAIL))
            store(pl.ds(NTILE * RPT, TAIL), pl.ds(0, TAIL))

    def run(h_hbm, out_hbm):
        def load_h(rng, sub):
            dstv = wb if sub is None else wb.at[sub]
            pltpu.sync_copy(h_hbm.at[rng], dstv)

        def store_acc(rng, sub):
            srcv = wb if sub is None else wb.at[sub]
            pltpu.sync_copy(srcv, acc.at[rng])

        slab_copy(load_h, store_acc)
        plsc.subcore_barrier()
        for g in range(CH // IDXG):
            pltpu.sync_copy(src_hbm.at[s, pl.ds(g * IDXG, IDXG)], sidx)
            pltpu.sync_copy(dst_hbm.at[s, pl.ds(g * IDXG, IDXG)], didx)
            for j in range(IDXG):
                pltpu.async_copy(h_hbm.at[sidx.at[j, 0]], rows, sem).wait()
                pltpu.sync_copy(rows, acc.at[didx.at[j, 0]], add=True)
        plsc.subcore_barrier()

        def load_acc(rng, sub):
            dstv = wb if sub is None else wb.at[sub]
            pltpu.sync_copy(acc.at[rng], dstv)

        def store_out(rng, sub):
            srcv = wb if sub is None else wb.at[sub]
            pltpu.sync_copy(srcv, out_hbm.at[rng])

        slab_copy(load_acc, store_out)

    @pl.when(c == 0)
    def _():
        run(h0_hbm, out0_hbm)

    @pl.when(c == 1)
    def _():
        run(h1_hbm, out1_hbm)


@jax.jit
def _sc_aggregate(h0, h1, src_r, dst_r):
    mesh = plsc.VectorSubcoreMesh(core_axis_name="c", subcore_axis_name="s")
    return pl.kernel(
        _sc_agg_body,
        mesh=mesh,
        out_type=(jax.ShapeDtypeStruct((N, HALF), jnp.float32),
                  jax.ShapeDtypeStruct((N, HALF), jnp.float32)),
        scratch_types=[
            pltpu.VMEM((IDXG, 1, K), jnp.int32),
            pltpu.VMEM((IDXG, 1, K), jnp.int32),
            pltpu.VMEM((K, HALF), jnp.float32),
            pltpu.VMEM((WB, HALF), jnp.float32),
            pltpu.VMEM_SHARED((N, HALF), jnp.float32),
            pltpu.SemaphoreType.DMA,
        ],
    )(h0, h1, src_r, dst_r)


def _mlp_body(a0_ref, a1_ref, w1_ref, b1_ref, w2_ref, b2_ref,
              u_ref, stats_ref, acc_ref):
    i = pl.program_id(0)
    z = jnp.concatenate([a0_ref[...], a1_ref[...]], axis=1)
    u = jnp.dot(z, w1_ref[...], preferred_element_type=jnp.float32) + b1_ref[...]
    u = jnp.maximum(u, 0.0)
    v = jnp.dot(u, w2_ref[...], preferred_element_type=jnp.float32) + b2_ref[...]
    v = jnp.maximum(v, 0.0)
    u_ref[...] = v
    blk = jnp.concatenate([jnp.sum(v, axis=0, keepdims=True),
                           jnp.sum(v * v, axis=0, keepdims=True)], axis=0)

    @pl.when(i == 0)
    def _():
        acc_ref[...] = blk

    @pl.when(i > 0)
    def _():
        acc_ref[...] += blk

    @pl.when(i == NB - 1)
    def _():
        stats_ref[...] = acc_ref[...]


@jax.jit
def _mlp(a0, a1, w1, b1, w2, b2):
    return pl.pallas_call(
        _mlp_body,
        grid=(NB,),
        in_specs=[
            pl.BlockSpec((BR, HALF), lambda i: (i, 0)),
            pl.BlockSpec((BR, HALF), lambda i: (i, 0)),
            pl.BlockSpec((DIM, DIM), lambda i: (0, 0)),
            pl.BlockSpec((1, DIM), lambda i: (0, 0)),
            pl.BlockSpec((DIM, DIM), lambda i: (0, 0)),
            pl.BlockSpec((1, DIM), lambda i: (0, 0)),
        ],
        out_specs=(
            pl.BlockSpec((BR, DIM), lambda i: (i, 0)),
            pl.BlockSpec((2, DIM), lambda i: (0, 0)),
        ),
        out_shape=(jax.ShapeDtypeStruct((N, DIM), jnp.float32),
                   jax.ShapeDtypeStruct((2, DIM), jnp.float32)),
        scratch_shapes=[pltpu.VMEM((2, DIM), jnp.float32)],
    )(a0, a1, w1, b1, w2, b2)


def _bn_pool_body(u_ref, stats_ref, g_ref, bt_ref, gi_ref,
                  z_ref, z0_ref, z1_ref, pool_ref, acc_ref):
    i = pl.program_id(0)
    mean = stats_ref[0:1, :] * (1.0 / N)
    var = stats_ref[1:2, :] * (1.0 / N) - mean * mean
    scale = g_ref[...] * lax.rsqrt(var + 1e-5)
    shift = bt_ref[...] - mean * scale
    z = u_ref[...] * scale + shift
    z_ref[...] = z
    z0_ref[...] = z[:, :HALF]
    z1_ref[...] = z[:, HALF:]
    ids = lax.broadcasted_iota(jnp.int32, (G, BR), 0)
    seg = (ids == gi_ref[0]).astype(jnp.float32)
    blk = jnp.dot(seg, z, preferred_element_type=jnp.float32,
                  precision=lax.Precision.HIGHEST)

    @pl.when(i == 0)
    def _():
        acc_ref[...] = blk

    @pl.when(i > 0)
    def _():
        acc_ref[...] += blk

    @pl.when(i == NB - 1)
    def _():
        pool_ref[...] = acc_ref[...]


@jax.jit
def _bn_pool(u, stats, gamma, beta, gi_r):
    return pl.pallas_call(
        _bn_pool_body,
        grid=(NB,),
        in_specs=[
            pl.BlockSpec((BR, DIM), lambda i: (i, 0)),
            pl.BlockSpec((2, DIM), lambda i: (0, 0)),
            pl.BlockSpec((1, DIM), lambda i: (0, 0)),
            pl.BlockSpec((1, DIM), lambda i: (0, 0)),
            pl.BlockSpec((1, 1, BR), lambda i: (i, 0, 0)),
        ],
        out_specs=(
            pl.BlockSpec((BR, DIM), lambda i: (i, 0)),
            pl.BlockSpec((BR, HALF), lambda i: (i, 0)),
            pl.BlockSpec((BR, HALF), lambda i: (i, 0)),
            pl.BlockSpec((G, DIM), lambda i: (0, 0)),
        ),
        out_shape=(jax.ShapeDtypeStruct((N, DIM), jnp.float32),
                   jax.ShapeDtypeStruct((N, HALF), jnp.float32),
                   jax.ShapeDtypeStruct((N, HALF), jnp.float32),
                   jax.ShapeDtypeStruct((G, DIM), jnp.float32)),
        scratch_shapes=[pltpu.VMEM((G, DIM), jnp.float32)],
    )(u, stats, gamma, beta, gi_r)


def kernel(node_features, edge_index, graph_index, params):
    src_r = edge_index[0].reshape(NTILE, CH, 1, K)
    dst_r = edge_index[1].reshape(NTILE, CH, 1, K)
    gi_r = graph_index.reshape(NB, 1, BR)
    h0 = node_features[:, :HALF]
    h1 = node_features[:, HALF:]
    zs, pools = [], []
    for p in params:
        zp0, zp1 = _sc_aggregate(h0, h1, src_r, dst_r)
        u, stats = _mlp(zp0, zp1, p["W1"], p["b1"].reshape(1, DIM),
                        p["W2"], p["b2"].reshape(1, DIM))
        z, z0, z1, pool = _bn_pool(u, stats, p["gamma"].reshape(1, DIM),
                                   p["beta"].reshape(1, DIM), gi_r)
        zs.append(z)
        pools.append(pool)
        h0, h1 = z0, z1
    return jnp.concatenate(pools, axis=1), jnp.concatenate(zs, axis=1)


# speedup vs baseline: 5.3859x; 1.0048x over previous
import functools

import jax
import jax.numpy as jnp
from jax import lax
from jax.experimental import pallas as pl
from jax.experimental.pallas import tpu as pltpu
from jax.experimental.pallas import tpu_sc as plsc

N = 10000
E = 160000
DIM = 256
HALF = 128
G = 64

NTILE = 16
EPT = E // NTILE
K = 80
CH = EPT // K
GC = 25
NG = CH // GC
RPT = 624
TAIL = N - NTILE * RPT

NB = 10
BR = N // NB


def _sc_agg_body(h0_hbm, h1_hbm, src_hbm, dst_hbm, out0_hbm, out1_hbm,
                 sidx, didx, buf0, buf1, acc, sem, isem):
    c = lax.axis_index("c")
    s = lax.axis_index("s")
    base = s * RPT

    def slab_copy(load, store):
        for kk in range(7):
            r0 = base + kk * K
            load(pl.ds(r0, K), buf0)
            store(pl.ds(r0, K), buf0)
        r0 = base + 7 * K
        load(pl.ds(r0, 64), buf0.at[pl.ds(0, 64)])
        store(pl.ds(r0, 64), buf0.at[pl.ds(0, 64)])

        @pl.when(s == NTILE - 1)
        def _():
            load(pl.ds(NTILE * RPT, TAIL), buf0.at[pl.ds(0, TAIL)])
            store(pl.ds(NTILE * RPT, TAIL), buf0.at[pl.ds(0, TAIL)])

    def run(h_hbm, out_hbm):
        def g_fire(p, j, buf):
            pltpu.async_copy(h_hbm.at[sidx.at[p, j, 0]], buf, sem)

        def g_wait(buf):
            pltpu.make_async_copy(h_hbm.at[sidx.at[0, 0, 0]], buf, sem).wait()

        def s_sync(p, j, buf):
            pltpu.sync_copy(buf, acc.at[didx.at[p, j, 0]], add=True)

        def idx_fire(g, p):
            pltpu.async_copy(src_hbm.at[s, pl.ds(g * GC, GC)], sidx.at[p], isem)
            pltpu.async_copy(dst_hbm.at[s, pl.ds(g * GC, GC)], didx.at[p], isem)

        def idx_wait():
            pltpu.make_async_copy(src_hbm.at[s, pl.ds(0, GC)], sidx.at[0], isem).wait()
            pltpu.make_async_copy(src_hbm.at[s, pl.ds(0, GC)], didx.at[0], isem).wait()

        slab_copy(lambda rng, b: pltpu.sync_copy(h_hbm.at[rng], b),
                  lambda rng, b: pltpu.sync_copy(b, acc.at[rng]))
        idx_fire(0, 0)
        idx_wait()
        plsc.subcore_barrier()

        for g in range(NG):
            p = g % 2
            if g + 1 < NG:
                idx_fire(g + 1, 1 - p)
            g_fire(p, 0, buf0)
            g_wait(buf0)

            def body(i, carry, p=p):
                j = 2 * i
                g_fire(p, j + 1, buf1)
                s_sync(p, j, buf0)
                g_wait(buf1)
                g_fire(p, j + 2, buf0)
                s_sync(p, j + 1, buf1)
                g_wait(buf0)
                return carry

            lax.fori_loop(0, (GC - 3) // 2, body, 0)
            g_fire(p, GC - 2, buf1)
            s_sync(p, GC - 3, buf0)
            g_wait(buf1)
            g_fire(p, GC - 1, buf0)
            s_sync(p, GC - 2, buf1)
            g_wait(buf0)
            s_sync(p, GC - 1, buf0)
            if g + 1 < NG:
                idx_wait()
        plsc.subcore_barrier()

        slab_copy(lambda rng, b: pltpu.sync_copy(acc.at[rng], b),
                  lambda rng, b: pltpu.sync_copy(b, out_hbm.at[rng]))

    @pl.when(c == 0)
    def _():
        run(h0_hbm, out0_hbm)

    @pl.when(c == 1)
    def _():
        run(h1_hbm, out1_hbm)


@jax.jit
def _sc_aggregate(h0, h1, src_r, dst_r):
    mesh = plsc.VectorSubcoreMesh(core_axis_name="c", subcore_axis_name="s")
    return pl.kernel(
        _sc_agg_body,
        mesh=mesh,
        out_type=(jax.ShapeDtypeStruct((N, HALF), jnp.float32),
                  jax.ShapeDtypeStruct((N, HALF), jnp.float32)),
        scratch_types=[
            pltpu.VMEM((2, GC, 1, K), jnp.int32),
            pltpu.VMEM((2, GC, 1, K), jnp.int32),
            pltpu.VMEM((K, HALF), jnp.float32),
            pltpu.VMEM((K, HALF), jnp.float32),
            pltpu.VMEM_SHARED((N, HALF), jnp.float32),
            pltpu.SemaphoreType.DMA,
            pltpu.SemaphoreType.DMA,
        ],
    )(h0, h1, src_r, dst_r)


def _mlp_body(a0_ref, a1_ref, w1_ref, b1_ref, w2_ref, b2_ref,
              u_ref, stats_ref, acc_ref):
    i = pl.program_id(0)
    z = jnp.concatenate([a0_ref[...], a1_ref[...]], axis=1)
    u = jnp.dot(z, w1_ref[...], preferred_element_type=jnp.float32) + b1_ref[...]
    u = jnp.maximum(u, 0.0)
    v = jnp.dot(u, w2_ref[...], preferred_element_type=jnp.float32) + b2_ref[...]
    v = jnp.maximum(v, 0.0)
    u_ref[...] = v
    blk = jnp.concatenate([jnp.sum(v, axis=0, keepdims=True),
                           jnp.sum(v * v, axis=0, keepdims=True)], axis=0)

    @pl.when(i == 0)
    def _():
        acc_ref[...] = blk

    @pl.when(i > 0)
    def _():
        acc_ref[...] += blk

    @pl.when(i == NB - 1)
    def _():
        stats_ref[...] = acc_ref[...]


@jax.jit
def _mlp(a0, a1, w1, b1, w2, b2):
    return pl.pallas_call(
        _mlp_body,
        grid=(NB,),
        in_specs=[
            pl.BlockSpec((BR, HALF), lambda i: (i, 0)),
            pl.BlockSpec((BR, HALF), lambda i: (i, 0)),
            pl.BlockSpec((DIM, DIM), lambda i: (0, 0)),
            pl.BlockSpec((1, DIM), lambda i: (0, 0)),
            pl.BlockSpec((DIM, DIM), lambda i: (0, 0)),
            pl.BlockSpec((1, DIM), lambda i: (0, 0)),
        ],
        out_specs=(
            pl.BlockSpec((BR, DIM), lambda i: (i, 0)),
            pl.BlockSpec((2, DIM), lambda i: (0, 0)),
        ),
        out_shape=(jax.ShapeDtypeStruct((N, DIM), jnp.float32),
                   jax.ShapeDtypeStruct((2, DIM), jnp.float32)),
        scratch_shapes=[pltpu.VMEM((2, DIM), jnp.float32)],
    )(a0, a1, w1, b1, w2, b2)


def _bn_pool_body(u_ref, stats_ref, g_ref, bt_ref, gi_ref,
                  z_ref, z0_ref, z1_ref, pool_ref, acc_ref):
    i = pl.program_id(0)
    mean = stats_ref[0:1, :] * (1.0 / N)
    var = stats_ref[1:2, :] * (1.0 / N) - mean * mean
    scale = g_ref[...] * lax.rsqrt(var + 1e-5)
    shift = bt_ref[...] - mean * scale
    z = u_ref[...] * scale + shift
    z_ref[...] = z
    z0_ref[...] = z[:, :HALF]
    z1_ref[...] = z[:, HALF:]
    ids = lax.broadcasted_iota(jnp.int32, (G, BR), 0)
    seg = (ids == gi_ref[0]).astype(jnp.float32)
    blk = jnp.dot(seg, z, preferred_element_type=jnp.float32,
                  precision=lax.Precision.HIGHEST)

    @pl.when(i == 0)
    def _():
        acc_ref[...] = blk

    @pl.when(i > 0)
    def _():
        acc_ref[...] += blk

    @pl.when(i == NB - 1)
    def _():
        pool_ref[...] = acc_ref[...]


@jax.jit
def _bn_pool(u, stats, gamma, beta, gi_r):
    return pl.pallas_call(
        _bn_pool_body,
        grid=(NB,),
        in_specs=[
            pl.BlockSpec((BR, DIM), lambda i: (i, 0)),
            pl.BlockSpec((2, DIM), lambda i: (0, 0)),
            pl.BlockSpec((1, DIM), lambda i: (0, 0)),
            pl.BlockSpec((1, DIM), lambda i: (0, 0)),
            pl.BlockSpec((1, 1, BR), lambda i: (i, 0, 0)),
        ],
        out_specs=(
            pl.BlockSpec((BR, DIM), lambda i: (i, 0)),
            pl.BlockSpec((BR, HALF), lambda i: (i, 0)),
            pl.BlockSpec((BR, HALF), lambda i: (i, 0)),
            pl.BlockSpec((G, DIM), lambda i: (0, 0)),
        ),
        out_shape=(jax.ShapeDtypeStruct((N, DIM), jnp.float32),
                   jax.ShapeDtypeStruct((N, HALF), jnp.float32),
                   jax.ShapeDtypeStruct((N, HALF), jnp.float32),
                   jax.ShapeDtypeStruct((G, DIM), jnp.float32)),
        scratch_shapes=[pltpu.VMEM((G, DIM), jnp.float32)],
    )(u, stats, gamma, beta, gi_r)


def kernel(node_features, edge_index, graph_index, params):
    src_r = edge_index[0].reshape(NTILE, CH, 1, K)
    dst_r = edge_index[1].reshape(NTILE, CH, 1, K)
    gi_r = graph_index.reshape(NB, 1, BR)
    h0 = node_features[:, :HALF]
    h1 = node_features[:, HALF:]
    zs, pools = [], []
    for p in params:
        zp0, zp1 = _sc_aggregate(h0, h1, src_r, dst_r)
        u, stats = _mlp(zp0, zp1, p["W1"], p["b1"].reshape(1, DIM),
                        p["W2"], p["b2"].reshape(1, DIM))
        z, z0, z1, pool = _bn_pool(u, stats, p["gamma"].reshape(1, DIM),
                                   p["beta"].reshape(1, DIM), gi_r)
        zs.append(z)
        pools.append(pool)
        h0, h1 = z0, z1
    return jnp.concatenate(pools, axis=1), jnp.concatenate(zs, axis=1)


# speedup vs baseline: 5.5142x; 1.0238x over previous
import functools

import jax
import jax.numpy as jnp
from jax import lax
from jax.experimental import pallas as pl
from jax.experimental.pallas import tpu as pltpu
from jax.experimental.pallas import tpu_sc as plsc

N = 10000
E = 160000
DIM = 256
HALF = 128
G = 64

NTILE = 16
EPT = E // NTILE
K = 80
CH = EPT // K
GC = 25
NG = CH // GC
RPT = 624
TAIL = N - NTILE * RPT

NB = 10
BR = N // NB


def _sc_agg_body(h0_hbm, h1_hbm, src_hbm, dst_hbm, out0_hbm, out1_hbm,
                 sidx, didx, buf0, buf1, acc, sem, isem):
    c = lax.axis_index("c")
    s = lax.axis_index("s")
    base = s * RPT

    def slab_copy(copy):
        copy(pl.ds(base, RPT))

        @pl.when(s == NTILE - 1)
        def _():
            copy(pl.ds(NTILE * RPT, TAIL))

    def run(h_hbm, out_hbm):
        def g_fire(p, j, buf):
            pltpu.async_copy(h_hbm.at[sidx.at[p, j, 0]], buf, sem)

        def g_wait(buf):
            pltpu.make_async_copy(h_hbm.at[sidx.at[0, 0, 0]], buf, sem).wait()

        def s_sync(p, j, buf):
            pltpu.sync_copy(buf, acc.at[didx.at[p, j, 0]], add=True)

        def idx_fire(g, p):
            pltpu.async_copy(src_hbm.at[s, pl.ds(g * GC, GC)], sidx.at[p], isem)
            pltpu.async_copy(dst_hbm.at[s, pl.ds(g * GC, GC)], didx.at[p], isem)

        def idx_wait():
            pltpu.make_async_copy(src_hbm.at[s, pl.ds(0, GC)], sidx.at[0], isem).wait()
            pltpu.make_async_copy(src_hbm.at[s, pl.ds(0, GC)], didx.at[0], isem).wait()

        slab_copy(lambda rng: pltpu.sync_copy(h_hbm.at[rng], acc.at[rng]))
        idx_fire(0, 0)
        idx_wait()
        plsc.subcore_barrier()

        for g in range(NG):
            p = g % 2
            if g + 1 < NG:
                idx_fire(g + 1, 1 - p)
            g_fire(p, 0, buf0)
            g_wait(buf0)

            def body(i, carry, p=p):
                j = 2 * i
                g_fire(p, j + 1, buf1)
                s_sync(p, j, buf0)
                g_wait(buf1)
                g_fire(p, j + 2, buf0)
                s_sync(p, j + 1, buf1)
                g_wait(buf0)
                return carry

            lax.fori_loop(0, (GC - 3) // 2, body, 0)
            g_fire(p, GC - 2, buf1)
            s_sync(p, GC - 3, buf0)
            g_wait(buf1)
            g_fire(p, GC - 1, buf0)
            s_sync(p, GC - 2, buf1)
            g_wait(buf0)
            s_sync(p, GC - 1, buf0)
            if g + 1 < NG:
                idx_wait()
        plsc.subcore_barrier()

        slab_copy(lambda rng: pltpu.sync_copy(acc.at[rng], out_hbm.at[rng]))

    @pl.when(c == 0)
    def _():
        run(h0_hbm, out0_hbm)

    @pl.when(c == 1)
    def _():
        run(h1_hbm, out1_hbm)


@jax.jit
def _sc_aggregate(h0, h1, src_r, dst_r):
    mesh = plsc.VectorSubcoreMesh(core_axis_name="c", subcore_axis_name="s")
    return pl.kernel(
        _sc_agg_body,
        mesh=mesh,
        out_type=(jax.ShapeDtypeStruct((N, HALF), jnp.float32),
                  jax.ShapeDtypeStruct((N, HALF), jnp.float32)),
        scratch_types=[
            pltpu.VMEM((2, GC, 1, K), jnp.int32),
            pltpu.VMEM((2, GC, 1, K), jnp.int32),
            pltpu.VMEM((K, HALF), jnp.float32),
            pltpu.VMEM((K, HALF), jnp.float32),
            pltpu.VMEM_SHARED((N, HALF), jnp.float32),
            pltpu.SemaphoreType.DMA,
            pltpu.SemaphoreType.DMA,
        ],
    )(h0, h1, src_r, dst_r)


def _mlp_body(a0_ref, a1_ref, w1_ref, b1_ref, w2_ref, b2_ref,
              u_ref, stats_ref, acc_ref):
    i = pl.program_id(0)
    z = jnp.concatenate([a0_ref[...], a1_ref[...]], axis=1)
    u = jnp.dot(z, w1_ref[...], preferred_element_type=jnp.float32) + b1_ref[...]
    u = jnp.maximum(u, 0.0)
    v = jnp.dot(u, w2_ref[...], preferred_element_type=jnp.float32) + b2_ref[...]
    v = jnp.maximum(v, 0.0)
    u_ref[...] = v
    blk = jnp.concatenate([jnp.sum(v, axis=0, keepdims=True),
                           jnp.sum(v * v, axis=0, keepdims=True)], axis=0)

    @pl.when(i == 0)
    def _():
        acc_ref[...] = blk

    @pl.when(i > 0)
    def _():
        acc_ref[...] += blk

    @pl.when(i == NB - 1)
    def _():
        stats_ref[...] = acc_ref[...]


@jax.jit
def _mlp(a0, a1, w1, b1, w2, b2):
    return pl.pallas_call(
        _mlp_body,
        grid=(NB,),
        in_specs=[
            pl.BlockSpec((BR, HALF), lambda i: (i, 0)),
            pl.BlockSpec((BR, HALF), lambda i: (i, 0)),
            pl.BlockSpec((DIM, DIM), lambda i: (0, 0)),
            pl.BlockSpec((1, DIM), lambda i: (0, 0)),
            pl.BlockSpec((DIM, DIM), lambda i: (0, 0)),
            pl.BlockSpec((1, DIM), lambda i: (0, 0)),
        ],
        out_specs=(
            pl.BlockSpec((BR, DIM), lambda i: (i, 0)),
            pl.BlockSpec((2, DIM), lambda i: (0, 0)),
        ),
        out_shape=(jax.ShapeDtypeStruct((N, DIM), jnp.float32),
                   jax.ShapeDtypeStruct((2, DIM), jnp.float32)),
        scratch_shapes=[pltpu.VMEM((2, DIM), jnp.float32)],
    )(a0, a1, w1, b1, w2, b2)


def _bn_pool_body(u_ref, stats_ref, g_ref, bt_ref, gi_ref,
                  z_ref, z0_ref, z1_ref, pool_ref, acc_ref):
    i = pl.program_id(0)
    mean = stats_ref[0:1, :] * (1.0 / N)
    var = stats_ref[1:2, :] * (1.0 / N) - mean * mean
    scale = g_ref[...] * lax.rsqrt(var + 1e-5)
    shift = bt_ref[...] - mean * scale
    z = u_ref[...] * scale + shift
    z_ref[...] = z
    z0_ref[...] = z[:, :HALF]
    z1_ref[...] = z[:, HALF:]
    ids = lax.broadcasted_iota(jnp.int32, (G, BR), 0)
    seg = (ids == gi_ref[0]).astype(jnp.float32)
    blk = jnp.dot(seg, z, preferred_element_type=jnp.float32,
                  precision=lax.Precision.HIGHEST)

    @pl.when(i == 0)
    def _():
        acc_ref[...] = blk

    @pl.when(i > 0)
    def _():
        acc_ref[...] += blk

    @pl.when(i == NB - 1)
    def _():
        pool_ref[...] = acc_ref[...]


@jax.jit
def _bn_pool(u, stats, gamma, beta, gi_r):
    return pl.pallas_call(
        _bn_pool_body,
        grid=(NB,),
        in_specs=[
            pl.BlockSpec((BR, DIM), lambda i: (i, 0)),
            pl.BlockSpec((2, DIM), lambda i: (0, 0)),
            pl.BlockSpec((1, DIM), lambda i: (0, 0)),
            pl.BlockSpec((1, DIM), lambda i: (0, 0)),
            pl.BlockSpec((1, 1, BR), lambda i: (i, 0, 0)),
        ],
        out_specs=(
            pl.BlockSpec((BR, DIM), lambda i: (i, 0)),
            pl.BlockSpec((BR, HALF), lambda i: (i, 0)),
            pl.BlockSpec((BR, HALF), lambda i: (i, 0)),
            pl.BlockSpec((G, DIM), lambda i: (0, 0)),
        ),
        out_shape=(jax.ShapeDtypeStruct((N, DIM), jnp.float32),
                   jax.ShapeDtypeStruct((N, HALF), jnp.float32),
                   jax.ShapeDtypeStruct((N, HALF), jnp.float32),
                   jax.ShapeDtypeStruct((G, DIM), jnp.float32)),
        scratch_shapes=[pltpu.VMEM((G, DIM), jnp.float32)],
    )(u, stats, gamma, beta, gi_r)


def kernel(node_features, edge_index, graph_index, params):
    src_r = edge_index[0].reshape(NTILE, CH, 1, K)
    dst_r = edge_index[1].reshape(NTILE, CH, 1, K)
    gi_r = graph_index.reshape(NB, 1, BR)
    h0 = node_features[:, :HALF]
    h1 = node_features[:, HALF:]
    zs, pools = [], []
    for p in params:
        zp0, zp1 = _sc_aggregate(h0, h1, src_r, dst_r)
        u, stats = _mlp(zp0, zp1, p["W1"], p["b1"].reshape(1, DIM),
                        p["W2"], p["b2"].reshape(1, DIM))
        z, z0, z1, pool = _bn_pool(u, stats, p["gamma"].reshape(1, DIM),
                                   p["beta"].reshape(1, DIM), gi_r)
        zs.append(z)
        pools.append(pool)
        h0, h1 = z0, z1
    return jnp.concatenate(pools, axis=1), jnp.concatenate(zs, axis=1)


# speedup vs baseline: 5.5328x; 1.0034x over previous
import functools

import jax
import jax.numpy as jnp
from jax import lax
from jax.experimental import pallas as pl
from jax.experimental.pallas import tpu as pltpu
from jax.experimental.pallas import tpu_sc as plsc

N = 10000
E = 160000
DIM = 256
HALF = 128
G = 64

NTILE = 16
EPT = E // NTILE
K = 80
CH = EPT // K
GC = 25
NG = CH // GC
RPT = 624
TAIL = N - NTILE * RPT

NB = 10
BR = N // NB


def _sc_agg_body(h0_hbm, h1_hbm, src_hbm, dst_hbm, out0_hbm, out1_hbm,
                 sidx, didx, buf0, buf1, acc, sem, ssem, isem):
    c = lax.axis_index("c")
    s = lax.axis_index("s")
    base = s * RPT

    def slab_copy(copy):
        copy(pl.ds(base, RPT))

        @pl.when(s == NTILE - 1)
        def _():
            copy(pl.ds(NTILE * RPT, TAIL))

    def run(h_hbm, out_hbm):
        def g_fire(p, j, buf):
            pltpu.async_copy(h_hbm.at[sidx.at[p, j, 0]], buf, sem)

        def g_wait(buf):
            pltpu.make_async_copy(h_hbm.at[sidx.at[0, 0, 0]], buf, sem).wait()

        def s_fire(p, j, buf):
            pltpu.async_copy(buf, acc.at[didx.at[p, j, 0]], ssem, add=True)

        def s_wait(buf):
            pltpu.make_async_copy(buf, acc.at[didx.at[0, 0, 0]], ssem).wait()

        def idx_fire(g, p):
            pltpu.async_copy(src_hbm.at[s, pl.ds(g * GC, GC)], sidx.at[p], isem)
            pltpu.async_copy(dst_hbm.at[s, pl.ds(g * GC, GC)], didx.at[p], isem)

        def idx_wait():
            pltpu.make_async_copy(src_hbm.at[s, pl.ds(0, GC)], sidx.at[0], isem).wait()
            pltpu.make_async_copy(src_hbm.at[s, pl.ds(0, GC)], didx.at[0], isem).wait()

        slab_copy(lambda rng: pltpu.sync_copy(h_hbm.at[rng], acc.at[rng]))
        idx_fire(0, 0)
        idx_wait()
        plsc.subcore_barrier()

        for g in range(NG):
            p = g % 2
            if g + 1 < NG:
                idx_fire(g + 1, 1 - p)
            g_fire(p, 0, buf0)
            g_wait(buf0)
            g_fire(p, 1, buf1)
            s_fire(p, 0, buf0)

            def body(i, carry, p=p):
                j = 1 + 2 * i
                g_wait(buf1)
                s_wait(buf0)
                g_fire(p, j + 1, buf0)
                s_fire(p, j, buf1)
                g_wait(buf0)
                s_wait(buf1)
                g_fire(p, j + 2, buf1)
                s_fire(p, j + 1, buf0)
                return carry

            lax.fori_loop(0, (GC - 3) // 2, body, 0)
            g_wait(buf1)
            s_wait(buf0)
            g_fire(p, GC - 1, buf0)
            s_fire(p, GC - 2, buf1)
            g_wait(buf0)
            s_wait(buf1)
            s_fire(p, GC - 1, buf0)
            s_wait(buf0)
            if g + 1 < NG:
                idx_wait()
        plsc.subcore_barrier()

        slab_copy(lambda rng: pltpu.sync_copy(acc.at[rng], out_hbm.at[rng]))

    @pl.when(c == 0)
    def _():
        run(h0_hbm, out0_hbm)

    @pl.when(c == 1)
    def _():
        run(h1_hbm, out1_hbm)


@jax.jit
def _sc_aggregate(h0, h1, src_r, dst_r):
    mesh = plsc.VectorSubcoreMesh(core_axis_name="c", subcore_axis_name="s")
    return pl.kernel(
        _sc_agg_body,
        mesh=mesh,
        out_type=(jax.ShapeDtypeStruct((N, HALF), jnp.float32),
                  jax.ShapeDtypeStruct((N, HALF), jnp.float32)),
        scratch_types=[
            pltpu.VMEM((2, GC, 1, K), jnp.int32),
            pltpu.VMEM((2, GC, 1, K), jnp.int32),
            pltpu.VMEM((K, HALF), jnp.float32),
            pltpu.VMEM((K, HALF), jnp.float32),
            pltpu.VMEM_SHARED((N, HALF), jnp.float32),
            pltpu.SemaphoreType.DMA,
            pltpu.SemaphoreType.DMA,
            pltpu.SemaphoreType.DMA,
        ],
    )(h0, h1, src_r, dst_r)


def _mlp_body(a0_ref, a1_ref, w1_ref, b1_ref, w2_ref, b2_ref,
              u_ref, stats_ref, acc_ref):
    i = pl.program_id(0)
    z = jnp.concatenate([a0_ref[...], a1_ref[...]], axis=1)
    u = jnp.dot(z, w1_ref[...], preferred_element_type=jnp.float32) + b1_ref[...]
    u = jnp.maximum(u, 0.0)
    v = jnp.dot(u, w2_ref[...], preferred_element_type=jnp.float32) + b2_ref[...]
    v = jnp.maximum(v, 0.0)
    u_ref[...] = v
    blk = jnp.concatenate([jnp.sum(v, axis=0, keepdims=True),
                           jnp.sum(v * v, axis=0, keepdims=True)], axis=0)

    @pl.when(i == 0)
    def _():
        acc_ref[...] = blk

    @pl.when(i > 0)
    def _():
        acc_ref[...] += blk

    @pl.when(i == NB - 1)
    def _():
        stats_ref[...] = acc_ref[...]


@jax.jit
def _mlp(a0, a1, w1, b1, w2, b2):
    return pl.pallas_call(
        _mlp_body,
        grid=(NB,),
        in_specs=[
            pl.BlockSpec((BR, HALF), lambda i: (i, 0)),
            pl.BlockSpec((BR, HALF), lambda i: (i, 0)),
            pl.BlockSpec((DIM, DIM), lambda i: (0, 0)),
            pl.BlockSpec((1, DIM), lambda i: (0, 0)),
            pl.BlockSpec((DIM, DIM), lambda i: (0, 0)),
            pl.BlockSpec((1, DIM), lambda i: (0, 0)),
        ],
        out_specs=(
            pl.BlockSpec((BR, DIM), lambda i: (i, 0)),
            pl.BlockSpec((2, DIM), lambda i: (0, 0)),
        ),
        out_shape=(jax.ShapeDtypeStruct((N, DIM), jnp.float32),
                   jax.ShapeDtypeStruct((2, DIM), jnp.float32)),
        scratch_shapes=[pltpu.VMEM((2, DIM), jnp.float32)],
    )(a0, a1, w1, b1, w2, b2)


def _bn_pool_body(u_ref, stats_ref, g_ref, bt_ref, gi_ref,
                  z_ref, z0_ref, z1_ref, pool_ref, acc_ref):
    i = pl.program_id(0)
    mean = stats_ref[0:1, :] * (1.0 / N)
    var = stats_ref[1:2, :] * (1.0 / N) - mean * mean
    scale = g_ref[...] * lax.rsqrt(var + 1e-5)
    shift = bt_ref[...] - mean * scale
    z = u_ref[...] * scale + shift
    z_ref[...] = z
    z0_ref[...] = z[:, :HALF]
    z1_ref[...] = z[:, HALF:]
    ids = lax.broadcasted_iota(jnp.int32, (G, BR), 0)
    seg = (ids == gi_ref[0]).astype(jnp.float32)
    blk = jnp.dot(seg, z, preferred_element_type=jnp.float32,
                  precision=lax.Precision.HIGHEST)

    @pl.when(i == 0)
    def _():
        acc_ref[...] = blk

    @pl.when(i > 0)
    def _():
        acc_ref[...] += blk

    @pl.when(i == NB - 1)
    def _():
        pool_ref[...] = acc_ref[...]


@jax.jit
def _bn_pool(u, stats, gamma, beta, gi_r):
    return pl.pallas_call(
        _bn_pool_body,
        grid=(NB,),
        in_specs=[
            pl.BlockSpec((BR, DIM), lambda i: (i, 0)),
            pl.BlockSpec((2, DIM), lambda i: (0, 0)),
            pl.BlockSpec((1, DIM), lambda i: (0, 0)),
            pl.BlockSpec((1, DIM), lambda i: (0, 0)),
            pl.BlockSpec((1, 1, BR), lambda i: (i, 0, 0)),
        ],
        out_specs=(
            pl.BlockSpec((BR, DIM), lambda i: (i, 0)),
            pl.BlockSpec((BR, HALF), lambda i: (i, 0)),
            pl.BlockSpec((BR, HALF), lambda i: (i, 0)),
            pl.BlockSpec((G, DIM), lambda i: (0, 0)),
        ),
        out_shape=(jax.ShapeDtypeStruct((N, DIM), jnp.float32),
                   jax.ShapeDtypeStruct((N, HALF), jnp.float32),
                   jax.ShapeDtypeStruct((N, HALF), jnp.float32),
                   jax.ShapeDtypeStruct((G, DIM), jnp.float32)),
        scratch_shapes=[pltpu.VMEM((G, DIM), jnp.float32)],
    )(u, stats, gamma, beta, gi_r)


def kernel(node_features, edge_index, graph_index, params):
    src_r = edge_index[0].reshape(NTILE, CH, 1, K)
    dst_r = edge_index[1].reshape(NTILE, CH, 1, K)
    gi_r = graph_index.reshape(NB, 1, BR)
    h0 = node_features[:, :HALF]
    h1 = node_features[:, HALF:]
    zs, pools = [], []
    for p in params:
        zp0, zp1 = _sc_aggregate(h0, h1, src_r, dst_r)
        u, stats = _mlp(zp0, zp1, p["W1"], p["b1"].reshape(1, DIM),
                        p["W2"], p["b2"].reshape(1, DIM))
        z, z0, z1, pool = _bn_pool(u, stats, p["gamma"].reshape(1, DIM),
                                   p["beta"].reshape(1, DIM), gi_r)
        zs.append(z)
        pools.append(pool)
        h0, h1 = z0, z1
    return jnp.concatenate(pools, axis=1), jnp.concatenate(zs, axis=1)


# speedup vs baseline: 5.7344x; 1.0364x over previous
import functools

import jax
import jax.numpy as jnp
from jax import lax
from jax.experimental import pallas as pl
from jax.experimental.pallas import tpu as pltpu
from jax.experimental.pallas import tpu_sc as plsc

N = 10000
E = 160000
DIM = 256
HALF = 128
G = 64

NTILE = 16
EPT = E // NTILE
K = 80
CH = EPT // K
GC = 25
NG = CH // GC
RPT = 624
TAIL = N - NTILE * RPT

NB = 10
BR = N // NB


def _sc_agg_body(h0_hbm, h1_hbm, src_hbm, dst_hbm, out0_hbm, out1_hbm,
                 sidx, didx, buf0, buf1, acc, sem, ssem, isem):
    c = lax.axis_index("c")
    s = lax.axis_index("s")
    base = s * RPT

    def slab_copy(copy):
        copy(pl.ds(base, RPT))

        @pl.when(s == NTILE - 1)
        def _():
            copy(pl.ds(NTILE * RPT, TAIL))

    def run(h_hbm, out_hbm):
        def g_fire(p, j, buf):
            pltpu.async_copy(h_hbm.at[sidx.at[p, j, 0]], buf, sem)

        def g_wait(buf):
            pltpu.make_async_copy(h_hbm.at[sidx.at[0, 0, 0]], buf, sem).wait()

        def s_fire(p, j, buf):
            pltpu.async_copy(buf, acc.at[didx.at[p, j, 0]], ssem, add=True)

        def s_wait(buf):
            pltpu.make_async_copy(buf, acc.at[didx.at[0, 0, 0]], ssem).wait()

        def idx_fire(g, p):
            pltpu.async_copy(src_hbm.at[s, pl.ds(g * GC, GC)], sidx.at[p], isem)
            pltpu.async_copy(dst_hbm.at[s, pl.ds(g * GC, GC)], didx.at[p], isem)

        def idx_wait():
            pltpu.make_async_copy(src_hbm.at[s, pl.ds(0, GC)], sidx.at[0], isem).wait()
            pltpu.make_async_copy(src_hbm.at[s, pl.ds(0, GC)], didx.at[0], isem).wait()

        slab_copy(lambda rng: pltpu.sync_copy(h_hbm.at[rng], acc.at[rng]))
        idx_fire(0, 0)
        idx_wait()
        plsc.subcore_barrier()

        for g in range(NG):
            p = g % 2
            if g + 1 < NG:
                idx_fire(g + 1, 1 - p)
            g_fire(p, 0, buf0)
            g_wait(buf0)
            g_fire(p, 1, buf1)
            s_fire(p, 0, buf0)

            def body(i, carry, p=p):
                j = 1 + 2 * i
                g_wait(buf1)
                s_wait(buf0)
                g_fire(p, j + 1, buf0)
                s_fire(p, j, buf1)
                g_wait(buf0)
                s_wait(buf1)
                g_fire(p, j + 2, buf1)
                s_fire(p, j + 1, buf0)
                return carry

            lax.fori_loop(0, (GC - 3) // 2, body, 0)
            g_wait(buf1)
            s_wait(buf0)
            g_fire(p, GC - 1, buf0)
            s_fire(p, GC - 2, buf1)
            g_wait(buf0)
            s_wait(buf1)
            s_fire(p, GC - 1, buf0)
            s_wait(buf0)
            if g + 1 < NG:
                idx_wait()
        plsc.subcore_barrier()

        slab_copy(lambda rng: pltpu.sync_copy(acc.at[rng], out_hbm.at[rng]))

    @pl.when(c == 0)
    def _():
        run(h0_hbm, out0_hbm)

    @pl.when(c == 1)
    def _():
        run(h1_hbm, out1_hbm)


@jax.jit
def _sc_aggregate(h0, h1, src_r, dst_r):
    mesh = plsc.VectorSubcoreMesh(core_axis_name="c", subcore_axis_name="s")
    return pl.kernel(
        _sc_agg_body,
        mesh=mesh,
        out_type=(jax.ShapeDtypeStruct((N, HALF), jnp.float32),
                  jax.ShapeDtypeStruct((N, HALF), jnp.float32)),
        scratch_types=[
            pltpu.VMEM((2, GC, 1, K), jnp.int32),
            pltpu.VMEM((2, GC, 1, K), jnp.int32),
            pltpu.VMEM((K, HALF), jnp.float32),
            pltpu.VMEM((K, HALF), jnp.float32),
            pltpu.VMEM_SHARED((N, HALF), jnp.float32),
            pltpu.SemaphoreType.DMA,
            pltpu.SemaphoreType.DMA,
            pltpu.SemaphoreType.DMA,
        ],
    )(h0, h1, src_r, dst_r)


def _tc_body(a0_ref, a1_ref, w1_ref, b1_ref, w2_ref, b2_ref, g_ref, bt_ref,
             gi_ref, z_ref, z0_ref, z1_ref, pool_ref, u_scr, st_scr, pl_scr):
    p = pl.program_id(0)
    i = pl.program_id(1)

    @pl.when(p == 0)
    def _():
        z = jnp.concatenate([a0_ref[...], a1_ref[...]], axis=1)
        u = jnp.dot(z, w1_ref[...], preferred_element_type=jnp.float32) + b1_ref[...]
        u = jnp.maximum(u, 0.0)
        v = jnp.dot(u, w2_ref[...], preferred_element_type=jnp.float32) + b2_ref[...]
        v = jnp.maximum(v, 0.0)
        u_scr[pl.ds(i * BR, BR), :] = v
        blk = jnp.concatenate([jnp.sum(v, axis=0, keepdims=True),
                               jnp.sum(v * v, axis=0, keepdims=True)], axis=0)

        @pl.when(i == 0)
        def _():
            st_scr[...] = blk

        @pl.when(i > 0)
        def _():
            st_scr[...] += blk

    @pl.when(p == 1)
    def _():
        mean = st_scr[0:1, :] * (1.0 / N)
        var = st_scr[1:2, :] * (1.0 / N) - mean * mean
        scale = g_ref[...] * lax.rsqrt(var + 1e-5)
        shift = bt_ref[...] - mean * scale
        z = u_scr[pl.ds(i * BR, BR), :] * scale + shift
        z_ref[...] = z
        z0_ref[...] = z[:, :HALF]
        z1_ref[...] = z[:, HALF:]
        ids = lax.broadcasted_iota(jnp.int32, (G, BR), 0)
        seg = (ids == gi_ref[0]).astype(jnp.float32)
        blk = jnp.dot(seg, z, preferred_element_type=jnp.float32,
                      precision=lax.Precision.HIGHEST)

        @pl.when(i == 0)
        def _():
            pl_scr[...] = blk

        @pl.when(i > 0)
        def _():
            pl_scr[...] += blk

        @pl.when(i == NB - 1)
        def _():
            pool_ref[...] = pl_scr[...]


@jax.jit
def _tc_layer(a0, a1, w1, b1, w2, b2, gamma, beta, gi_r):
    return pl.pallas_call(
        _tc_body,
        grid=(2, NB),
        in_specs=[
            pl.BlockSpec((BR, HALF), lambda p, i: ((1 - p) * i, 0)),
            pl.BlockSpec((BR, HALF), lambda p, i: ((1 - p) * i, 0)),
            pl.BlockSpec((DIM, DIM), lambda p, i: (0, 0)),
            pl.BlockSpec((1, DIM), lambda p, i: (0, 0)),
            pl.BlockSpec((DIM, DIM), lambda p, i: (0, 0)),
            pl.BlockSpec((1, DIM), lambda p, i: (0, 0)),
            pl.BlockSpec((1, DIM), lambda p, i: (0, 0)),
            pl.BlockSpec((1, DIM), lambda p, i: (0, 0)),
            pl.BlockSpec((1, 1, BR), lambda p, i: (p * i, 0, 0)),
        ],
        out_specs=(
            pl.BlockSpec((BR, DIM), lambda p, i: (p * i, 0)),
            pl.BlockSpec((BR, HALF), lambda p, i: (p * i, 0)),
            pl.BlockSpec((BR, HALF), lambda p, i: (p * i, 0)),
            pl.BlockSpec((G, DIM), lambda p, i: (0, 0)),
        ),
        out_shape=(jax.ShapeDtypeStruct((N, DIM), jnp.float32),
                   jax.ShapeDtypeStruct((N, HALF), jnp.float32),
                   jax.ShapeDtypeStruct((N, HALF), jnp.float32),
                   jax.ShapeDtypeStruct((G, DIM), jnp.float32)),
        scratch_shapes=[pltpu.VMEM((N, DIM), jnp.float32),
                        pltpu.VMEM((2, DIM), jnp.float32),
                        pltpu.VMEM((G, DIM), jnp.float32)],
    )(a0, a1, w1, b1, w2, b2, gamma, beta, gi_r)


def kernel(node_features, edge_index, graph_index, params):
    src_r = edge_index[0].reshape(NTILE, CH, 1, K)
    dst_r = edge_index[1].reshape(NTILE, CH, 1, K)
    gi_r = graph_index.reshape(NB, 1, BR)
    h0 = node_features[:, :HALF]
    h1 = node_features[:, HALF:]
    zs, pools = [], []
    for p in params:
        zp0, zp1 = _sc_aggregate(h0, h1, src_r, dst_r)
        z, z0, z1, pool = _tc_layer(zp0, zp1, p["W1"], p["b1"].reshape(1, DIM),
                                    p["W2"], p["b2"].reshape(1, DIM),
                                    p["gamma"].reshape(1, DIM),
                                    p["beta"].reshape(1, DIM), gi_r)
        zs.append(z)
        pools.append(pool)
        h0, h1 = z0, z1
    return jnp.concatenate(pools, axis=1), jnp.concatenate(zs, axis=1)


# speedup vs baseline: 5.7432x; 1.0015x over previous
import jax
import jax.numpy as jnp
from jax import lax
from jax.experimental import pallas as pl
from jax.experimental.pallas import tpu as pltpu
from jax.experimental.pallas import tpu_sc as plsc

N = 10000
E = 160000
DIM = 256
HALF = 128
G = 64

NTILE = 16
EPT = E // NTILE
K = 80
CH = EPT // K
GC = 25
NG = CH // GC
RPT = 624
TAIL = N - NTILE * RPT

NB = 10
BR = N // NB


def _sc_agg_body(h0_hbm, h1_hbm, src_hbm, dst_hbm, out0_hbm, out1_hbm,
                 sidx, didx, buf0, buf1, acc, sem, ssem, isem):
    c = lax.axis_index("c")
    s = lax.axis_index("s")
    base = s * RPT

    def slab_copy(copy):
        copy(pl.ds(base, RPT))

        @pl.when(s == NTILE - 1)
        def _():
            copy(pl.ds(NTILE * RPT, TAIL))

    def run(h_hbm, out_hbm):
        def g_fire(p, j, buf):
            pltpu.async_copy(h_hbm.at[sidx.at[p, j, 0]], buf, sem)

        def g_wait(buf):
            pltpu.make_async_copy(h_hbm.at[sidx.at[0, 0, 0]], buf, sem).wait()

        def s_fire(p, j, buf):
            pltpu.async_copy(buf, acc.at[didx.at[p, j, 0]], ssem, add=True)

        def s_wait(buf):
            pltpu.make_async_copy(buf, acc.at[didx.at[0, 0, 0]], ssem).wait()

        def idx_fire(g, p):
            pltpu.async_copy(src_hbm.at[s, pl.ds(g * GC, GC)], sidx.at[p], isem)
            pltpu.async_copy(dst_hbm.at[s, pl.ds(g * GC, GC)], didx.at[p], isem)

        def idx_wait():
            pltpu.make_async_copy(src_hbm.at[s, pl.ds(0, GC)], sidx.at[0], isem).wait()
            pltpu.make_async_copy(src_hbm.at[s, pl.ds(0, GC)], didx.at[0], isem).wait()

        slab_copy(lambda rng: pltpu.sync_copy(h_hbm.at[rng], acc.at[rng]))
        idx_fire(0, 0)
        idx_wait()
        plsc.subcore_barrier()

        for g in range(NG):
            p = g % 2
            if g + 1 < NG:
                idx_fire(g + 1, 1 - p)
            g_fire(p, 0, buf0)
            g_wait(buf0)
            g_fire(p, 1, buf1)
            s_fire(p, 0, buf0)

            def body(i, carry, p=p):
                j = 1 + 2 * i
                g_wait(buf1)
                s_wait(buf0)
                g_fire(p, j + 1, buf0)
                s_fire(p, j, buf1)
                g_wait(buf0)
                s_wait(buf1)
                g_fire(p, j + 2, buf1)
                s_fire(p, j + 1, buf0)
                return carry

            lax.fori_loop(0, (GC - 3) // 2, body, 0)
            g_wait(buf1)
            s_wait(buf0)
            g_fire(p, GC - 1, buf0)
            s_fire(p, GC - 2, buf1)
            g_wait(buf0)
            s_wait(buf1)
            s_fire(p, GC - 1, buf0)
            s_wait(buf0)
            if g + 1 < NG:
                idx_wait()
        plsc.subcore_barrier()

        slab_copy(lambda rng: pltpu.sync_copy(acc.at[rng], out_hbm.at[rng]))

    @pl.when(c == 0)
    def _():
        run(h0_hbm, out0_hbm)

    @pl.when(c == 1)
    def _():
        run(h1_hbm, out1_hbm)


@jax.jit
def _sc_aggregate(h0, h1, src_r, dst_r):
    mesh = plsc.VectorSubcoreMesh(core_axis_name="c", subcore_axis_name="s")
    return pl.kernel(
        _sc_agg_body,
        mesh=mesh,
        out_type=(jax.ShapeDtypeStruct((N, HALF), jnp.float32),
                  jax.ShapeDtypeStruct((N, HALF), jnp.float32)),
        scratch_types=[
            pltpu.VMEM((2, GC, 1, K), jnp.int32),
            pltpu.VMEM((2, GC, 1, K), jnp.int32),
            pltpu.VMEM((K, HALF), jnp.float32),
            pltpu.VMEM((K, HALF), jnp.float32),
            pltpu.VMEM_SHARED((N, HALF), jnp.float32),
            pltpu.SemaphoreType.DMA,
            pltpu.SemaphoreType.DMA,
            pltpu.SemaphoreType.DMA,
        ],
    )(h0, h1, src_r, dst_r)


def _tc_body(a0_ref, a1_ref, w1_ref, b1_ref, w2_ref, b2_ref, g_ref, bt_ref,
             gi_ref, z_ref, z0_ref, z1_ref, pool_ref, u_scr, st_scr, pl_scr):
    p = pl.program_id(0)
    i = pl.program_id(1)

    @pl.when(p == 0)
    def _():
        z = jnp.concatenate([a0_ref[...], a1_ref[...]], axis=1)
        u = jnp.dot(z, w1_ref[...], preferred_element_type=jnp.float32) + b1_ref[...]
        u = jnp.maximum(u, 0.0)
        v = jnp.dot(u, w2_ref[...], preferred_element_type=jnp.float32) + b2_ref[...]
        v = jnp.maximum(v, 0.0)
        u_scr[pl.ds(i * BR, BR), :] = v
        blk = jnp.concatenate([jnp.sum(v, axis=0, keepdims=True),
                               jnp.sum(v * v, axis=0, keepdims=True)], axis=0)

        @pl.when(i == 0)
        def _():
            st_scr[...] = blk

        @pl.when(i > 0)
        def _():
            st_scr[...] += blk

    @pl.when(p == 1)
    def _():
        mean = st_scr[0:1, :] * (1.0 / N)
        var = st_scr[1:2, :] * (1.0 / N) - mean * mean
        scale = g_ref[...] * lax.rsqrt(var + 1e-5)
        shift = bt_ref[...] - mean * scale
        z = u_scr[pl.ds(i * BR, BR), :] * scale + shift
        z_ref[...] = z
        z0_ref[...] = z[:, :HALF]
        z1_ref[...] = z[:, HALF:]
        ids = lax.broadcasted_iota(jnp.int32, (G, BR), 0)
        seg = (ids == gi_ref[0]).astype(jnp.float32)
        blk = jnp.dot(seg, z, preferred_element_type=jnp.float32,
                      precision=lax.Precision.HIGHEST)

        @pl.when(i == 0)
        def _():
            pl_scr[...] = blk

        @pl.when(i > 0)
        def _():
            pl_scr[...] += blk

        @pl.when(i == NB - 1)
        def _():
            pool_ref[...] = pl_scr[...]


@jax.jit
def _tc_layer(a0, a1, w1, b1, w2, b2, gamma, beta, gi_r):
    return pl.pallas_call(
        _tc_body,
        grid=(2, NB),
        in_specs=[
            pl.BlockSpec((BR, HALF), lambda p, i: ((1 - p) * i, 0)),
            pl.BlockSpec((BR, HALF), lambda p, i: ((1 - p) * i, 0)),
            pl.BlockSpec((DIM, DIM), lambda p, i: (0, 0)),
            pl.BlockSpec((1, DIM), lambda p, i: (0, 0)),
            pl.BlockSpec((DIM, DIM), lambda p, i: (0, 0)),
            pl.BlockSpec((1, DIM), lambda p, i: (0, 0)),
            pl.BlockSpec((1, DIM), lambda p, i: (0, 0)),
            pl.BlockSpec((1, DIM), lambda p, i: (0, 0)),
            pl.BlockSpec((1, 1, BR), lambda p, i: (p * i, 0, 0)),
        ],
        out_specs=(
            pl.BlockSpec((BR, DIM), lambda p, i: (p * i, 0)),
            pl.BlockSpec((BR, HALF), lambda p, i: (p * i, 0)),
            pl.BlockSpec((BR, HALF), lambda p, i: (p * i, 0)),
            pl.BlockSpec((G, DIM), lambda p, i: (0, 0)),
        ),
        out_shape=(jax.ShapeDtypeStruct((N, DIM), jnp.float32),
                   jax.ShapeDtypeStruct((N, HALF), jnp.float32),
                   jax.ShapeDtypeStruct((N, HALF), jnp.float32),
                   jax.ShapeDtypeStruct((G, DIM), jnp.float32)),
        scratch_shapes=[pltpu.VMEM((N, DIM), jnp.float32),
                        pltpu.VMEM((2, DIM), jnp.float32),
                        pltpu.VMEM((G, DIM), jnp.float32)],
    )(a0, a1, w1, b1, w2, b2, gamma, beta, gi_r)


def kernel(node_features, edge_index, graph_index, params):
    src_r = edge_index[0].reshape(NTILE, CH, 1, K)
    dst_r = edge_index[1].reshape(NTILE, CH, 1, K)
    gi_r = graph_index.reshape(NB, 1, BR)
    h0 = node_features[:, :HALF]
    h1 = node_features[:, HALF:]
    zs, pools = [], []
    for p in params:
        zp0, zp1 = _sc_aggregate(h0, h1, src_r, dst_r)
        z, z0, z1, pool = _tc_layer(zp0, zp1, p["W1"], p["b1"].reshape(1, DIM),
                                    p["W2"], p["b2"].reshape(1, DIM),
                                    p["gamma"].reshape(1, DIM),
                                    p["beta"].reshape(1, DIM), gi_r)
        zs.append(z)
        pools.append(pool)
        h0, h1 = z0, z1
    return jnp.concatenate(pools, axis=1), jnp.concatenate(zs, axis=1)
